# baseline (speedup 1.0000x reference)
import functools

import jax
import jax.numpy as jnp
from jax import lax
from jax.experimental import pallas as pl
from jax.experimental.pallas import tpu as pltpu
from jax.experimental.pallas import tpu_sc as plsc

N = 10000
E = 320000
G = 64
EPS = 1e-5
NC, NS = 2, 16
BM = 400
GRID = N // BM
K = 128
EA = 344064
NP = N + 16
STRIPE = 624


def _mesh():
    return plsc.VectorSubcoreMesh(
        core_axis_name="c", subcore_axis_name="s",
        num_cores=NC, num_subcores=NS)


def _init_zero(zero_hbm, zsh, s):
    r0 = s * STRIPE
    pltpu.sync_copy(zero_hbm.at[pl.ds(r0, STRIPE)], zsh.at[pl.ds(r0, STRIPE)])

    @pl.when(s == NS - 1)
    def _():
        t0 = NS * STRIPE
        pltpu.sync_copy(zero_hbm.at[pl.ds(t0, NP - t0)], zsh.at[pl.ds(t0, NP - t0)])


def _write_out(zsh, out_hbm, c, s):
    r0 = s * STRIPE
    pltpu.sync_copy(zsh.at[pl.ds(r0, STRIPE)], out_hbm.at[c, pl.ds(r0, STRIPE)])

    @pl.when(s == NS - 1)
    def _():
        t0 = NS * STRIPE
        pltpu.sync_copy(zsh.at[pl.ds(t0, N - t0)], out_hbm.at[c, pl.ds(t0, N - t0)])


def _make_deg():
    ept = EA // (NC * NS)
    nblk = ept // K

    @functools.partial(
        pl.kernel, mesh=_mesh(),
        out_type=jax.ShapeDtypeStruct((NC, N, 128), jnp.float32),
        scratch_types=[
            pltpu.VMEM((6, K), jnp.int32),
            pltpu.VMEM((K, 128), jnp.float32),
            pltpu.VMEM_SHARED((NP, 128), jnp.float32),
        ] + [pltpu.SemaphoreType.DMA] * 9,
    )
    def deg_kernel(dst_hbm, ones_hbm, zero_hbm, degz_hbm, idx_d, ones_v, zsh,
                   *sems):
        sem_i = sems[0:6]
        sem_s = sems[6:9]
        c = lax.axis_index("c")
        s = lax.axis_index("s")
        _init_zero(zero_hbm, zsh, s)
        pltpu.sync_copy(ones_hbm, ones_v)
        base = (s * NC + c) * ept

        def issue_idx(kk, ib):
            pltpu.async_copy(dst_hbm.at[pl.ds(base + kk * K, K)],
                             idx_d.at[ib], sem_i[ib])

        for ib in range(3):
            issue_idx(ib, ib)
        plsc.subcore_barrier()

        def turn(kk, u):
            b = u % 3
            pltpu.make_async_copy(dst_hbm.at[pl.ds(base, K)],
                                  idx_d.at[u], sem_i[u]).wait()
            pltpu.async_copy(ones_v, zsh.at[idx_d.at[u]], sem_s[b], add=True)

            @pl.when(kk >= 2)
            def _():
                u2, b2 = (u - 2) % 6, (u + 1) % 3
                pltpu.make_async_copy(ones_v, zsh.at[idx_d.at[u2]],
                                      sem_s[b2]).wait()

            @pl.when(kk + 3 < nblk)
            def _():
                issue_idx(kk + 3, (u + 3) % 6)

        def outer(j, carry):
            for u in range(6):
                turn(j * 6 + u, u)
            return carry

        lax.fori_loop(0, nblk // 6, outer, 0)
        for kk in (nblk - 2, nblk - 1):
            u, b = kk % 6, kk % 3
            pltpu.make_async_copy(ones_v, zsh.at[idx_d.at[u]], sem_s[b]).wait()
        plsc.subcore_barrier()
        _write_out(zsh, degz_hbm, c, s)

    return deg_kernel


def _make_scatter(feature_split):
    ept = EA // NS if feature_split else EA // (NC * NS)
    nblk = ept // K

    @functools.partial(
        pl.kernel, mesh=_mesh(),
        out_type=jax.ShapeDtypeStruct((NC, N, 128), jnp.float32),
        scratch_types=[
            pltpu.VMEM((6, K), jnp.int32),
            pltpu.VMEM((6, K), jnp.int32),
            pltpu.VMEM((2, K, 128), jnp.float32),
            pltpu.VMEM_SHARED((NP, 128), jnp.float32),
        ] + [pltpu.SemaphoreType.DMA] * 10,
    )
    def scat_kernel(src_hbm, dst_hbm, y_hbm, zero_hbm, z_hbm,
                    idx_s, idx_d, rows, zsh, *sems):
        sem_i = sems[0:6]
        sem_g = sems[6:8]
        sem_s = sems[8:10]
        c = lax.axis_index("c")
        s = lax.axis_index("s")
        _init_zero(zero_hbm, zsh, s)
        base = (s * ept) if feature_split else ((s * NC + c) * ept)

        def issue_idx(kk, ib):
            off = base + kk * K
            pltpu.async_copy(src_hbm.at[pl.ds(off, K)], idx_s.at[ib], sem_i[ib])
            pltpu.async_copy(dst_hbm.at[pl.ds(off, K)], idx_d.at[ib], sem_i[ib])

        for ib in range(3):
            issue_idx(ib, ib)
        plsc.subcore_barrier()

        def turn(kk, u):
            b = u % 2
            pltpu.make_async_copy(src_hbm.at[pl.ds(base, K)],
                                  idx_s.at[u], sem_i[u]).wait()
            pltpu.make_async_copy(dst_hbm.at[pl.ds(base, K)],
                                  idx_d.at[u], sem_i[u]).wait()
            if feature_split:
                for t in range(K // 16):
                    sl = pl.ds(t * 16, 16)
                    idx_s[u, sl] = idx_s[u, sl] * 2 + c

            @pl.when(kk >= 2)
            def _():
                u2 = (u - 2) % 6
                pltpu.make_async_copy(rows.at[b], zsh.at[idx_d.at[u2]],
                                      sem_s[b]).wait()

            pltpu.async_copy(y_hbm.at[idx_s.at[u]], rows.at[b], sem_g[b])

            @pl.when(kk >= 1)
            def _():
                up, bp = (u - 1) % 6, (u + 1) % 2
                pltpu.make_async_copy(y_hbm.at[idx_s.at[up]],
                                      rows.at[bp], sem_g[bp]).wait()
                pltpu.async_copy(rows.at[bp], zsh.at[idx_d.at[up]],
                                 sem_s[bp], add=True)

            @pl.when(kk + 3 < nblk)
            def _():
                issue_idx(kk + 3, (u + 3) % 6)

        def outer(j, carry):
            for u in range(6):
                turn(j * 6 + u, u)
            return carry

        lax.fori_loop(0, nblk // 6, outer, 0)
        ul, bl = (nblk - 1) % 6, (nblk - 1) % 2
        pltpu.make_async_copy(y_hbm.at[idx_s.at[ul]], rows.at[bl],
                              sem_g[bl]).wait()
        pltpu.async_copy(rows.at[bl], zsh.at[idx_d.at[ul]], sem_s[bl], add=True)
        for kk in (nblk - 2, nblk - 1):
            u, b = kk % 6, kk % 2
            pltpu.make_async_copy(rows.at[b], zsh.at[idx_d.at[u]],
                                  sem_s[b]).wait()
        plsc.subcore_barrier()
        _write_out(zsh, z_hbm, c, s)

    return scat_kernel


_DEG = _make_deg()
_SCAT_E = _make_scatter(False)
_SCAT_F = _make_scatter(True)


def _tc_l1(x, W1, degz):
    def body(x_ref, w_ref, dz_ref, y_ref, dinv_ref):
        dz = dz_ref[...]
        dinv = lax.rsqrt(dz[0][:, 0:1] + dz[1][:, 0:1])
        y_ref[...] = jnp.dot(x_ref[...], w_ref[...],
                             preferred_element_type=jnp.float32) * dinv
        dinv_ref[...] = jnp.broadcast_to(dinv, (BM, 8))

    return pl.pallas_call(
        body,
        grid=(GRID,),
        in_specs=[pl.BlockSpec((BM, 128), lambda i: (i, 0)),
                  pl.BlockSpec((128, 128), lambda i: (0, 0)),
                  pl.BlockSpec((2, BM, 128), lambda i: (0, i, 0))],
        out_specs=[pl.BlockSpec((BM, 128), lambda i: (i, 0)),
                   pl.BlockSpec((BM, 8), lambda i: (i, 0))],
        out_shape=[jax.ShapeDtypeStruct((N, 128), jnp.float32),
                   jax.ShapeDtypeStruct((N, 8), jnp.float32)],
    )(x, W1, degz)


def _tc_layer(z, dinv8, b, g, be, rm, rv, Wn, Wout):
    def body(z_ref, dinv_ref, b_ref, g_ref, be_ref, rm_ref, rv_ref,
             wn_ref, y_ref):
        dinv = dinv_ref[:, 0:1]
        pre = (z_ref[0] + z_ref[1]) * dinv + b_ref[...]
        scale = g_ref[...] * lax.rsqrt(rv_ref[...] + EPS)
        h = jnp.maximum((pre - rm_ref[...]) * scale + be_ref[...], 0.0)
        y_ref[...] = jnp.dot(h, wn_ref[...],
                             preferred_element_type=jnp.float32) * dinv

    pspec = pl.BlockSpec((1, 128), lambda i: (0, 0))
    return pl.pallas_call(
        body,
        grid=(GRID,),
        in_specs=[pl.BlockSpec((2, BM, 128), lambda i: (0, i, 0)),
                  pl.BlockSpec((BM, 8), lambda i: (i, 0)),
                  pspec, pspec, pspec, pspec, pspec,
                  pl.BlockSpec((128, Wout), lambda i: (0, 0))],
        out_specs=pl.BlockSpec((BM, Wout), lambda i: (i, 0)),
        out_shape=jax.ShapeDtypeStruct((N, Wout), jnp.float32),
    )(z, dinv8, b, g, be, rm, rv, Wn)


def _tc_final(z3, dinv8, b3, batch3):
    def body(z_ref, dinv_ref, b_ref, bat_ref, out_ref, acc0_ref, acc1_ref, cnt_ref):
        i = pl.program_id(0)

        @pl.when(i == 0)
        def _():
            acc0_ref[...] = jnp.zeros_like(acc0_ref)
            acc1_ref[...] = jnp.zeros_like(acc1_ref)
            cnt_ref[...] = jnp.zeros_like(cnt_ref)

        dinv = dinv_ref[:, 0:1]
        bidx = bat_ref[0, 0, :]
        oh = (bidx[:, None] == lax.broadcasted_iota(jnp.int32, (BM, G), 1)
              ).astype(jnp.float32)
        dn = (((0,), (0,)), ((), ()))
        for c, acc_ref in ((0, acc0_ref), (1, acc1_ref)):
            h = z_ref[c] * dinv + b_ref[c]
            acc_ref[...] += lax.dot_general(
                oh, h, dn, preferred_element_type=jnp.float32)
        cnt_ref[...] += lax.dot_general(
            oh, jnp.ones((BM, 128), jnp.float32), dn,
            preferred_element_type=jnp.float32)

        @pl.when(i == GRID - 1)
        def _():
            cnt = jnp.maximum(cnt_ref[:, 0:1], 1.0)
            out_ref[:, 0:128] = acc0_ref[...] / cnt
            out_ref[:, 128:256] = acc1_ref[...] / cnt

    return pl.pallas_call(
        body,
        grid=(GRID,),
        in_specs=[pl.BlockSpec((2, BM, 128), lambda i: (0, i, 0)),
                  pl.BlockSpec((BM, 8), lambda i: (i, 0)),
                  pl.BlockSpec((2, 1, 128), lambda i: (0, 0, 0)),
                  pl.BlockSpec((1, 1, BM), lambda i: (i, 0, 0))],
        out_specs=pl.BlockSpec((G, 256), lambda i: (0, 0)),
        out_shape=jax.ShapeDtypeStruct((G, 256), jnp.float32),
        scratch_shapes=[pltpu.VMEM((G, 128), jnp.float32),
                        pltpu.VMEM((G, 128), jnp.float32),
                        pltpu.VMEM((G, 128), jnp.float32)],
    )(z3, dinv8, b3, batch3)


def kernel(x, edge_index, batch, W1, b1, g1, be1, rm1, rv1,
           W2, b2, g2, be2, rm2, rv2, W3, b3):
    ei = edge_index.astype(jnp.int32)
    loops = jnp.arange(N, dtype=jnp.int32)
    pad = EA - E - N
    src_a = jnp.concatenate([ei[0], loops, jnp.zeros((pad,), jnp.int32)])
    dst_a = jnp.concatenate([ei[1], loops, jnp.full((pad,), N, jnp.int32)])
    batch3 = batch.astype(jnp.int32).reshape(GRID, 1, BM)
    ones_k = jnp.ones((K, 128), jnp.float32)
    zeros_np = jnp.zeros((NP, 128), jnp.float32)

    def row(a):
        return a.reshape(1, -1)

    degz = _DEG(dst_a, ones_k, zeros_np)
    y1, dinv8 = _tc_l1(x, W1, degz)
    z1 = _SCAT_E(src_a, dst_a, y1, zeros_np)
    y2 = _tc_layer(z1, dinv8, row(b1), row(g1), row(be1), row(rm1), row(rv1),
                   W2, 128)
    z2 = _SCAT_E(src_a, dst_a, y2, zeros_np)
    y3 = _tc_layer(z2, dinv8, row(b2), row(g2), row(be2), row(rm2), row(rv2),
                   W3, 256)
    z3 = _SCAT_F(src_a, dst_a, y3.reshape(2 * N, 128), zeros_np)
    return _tc_final(z3, dinv8, b3.reshape(2, 1, 128), batch3)


# speedup vs baseline: 1.8716x; 1.8716x over previous
import functools

import jax
import jax.numpy as jnp
from jax import lax
from jax.experimental import pallas as pl
from jax.experimental.pallas import tpu as pltpu
from jax.experimental.pallas import tpu_sc as plsc

N = 10000
E = 320000
G = 64
EPS = 1e-5
NC, NS = 2, 16
BM = 400
GRID = N // BM
K = 128
EA = 331776
NP = N + 16
STRIPE = 624


def _mesh():
    return plsc.VectorSubcoreMesh(
        core_axis_name="c", subcore_axis_name="s",
        num_cores=NC, num_subcores=NS)


def _init_zero(zero_hbm, zsh, s):
    r0 = s * STRIPE
    pltpu.sync_copy(zero_hbm.at[pl.ds(r0, STRIPE)], zsh.at[pl.ds(r0, STRIPE)])

    @pl.when(s == NS - 1)
    def _():
        t0 = NS * STRIPE
        pltpu.sync_copy(zero_hbm.at[pl.ds(t0, NP - t0)], zsh.at[pl.ds(t0, NP - t0)])


def _write_out(zsh, out_hbm, c, s):
    r0 = s * STRIPE
    pltpu.sync_copy(zsh.at[pl.ds(r0, STRIPE)], out_hbm.at[c, pl.ds(r0, STRIPE)])

    @pl.when(s == NS - 1)
    def _():
        t0 = NS * STRIPE
        pltpu.sync_copy(zsh.at[pl.ds(t0, N - t0)], out_hbm.at[c, pl.ds(t0, N - t0)])


def _make_deg():
    ept = EA // (NC * NS)
    nblk = ept // K

    @functools.partial(
        pl.kernel, mesh=_mesh(),
        out_type=jax.ShapeDtypeStruct((NC, N, 128), jnp.float32),
        scratch_types=[
            pltpu.VMEM((K,), jnp.int32),
            pltpu.VMEM((K, 128), jnp.float32),
            pltpu.VMEM_SHARED((NP, 128), jnp.float32),
        ],
    )
    def deg_kernel(dst_hbm, ones_hbm, zero_hbm, degz_hbm, idx_d, ones_v, zsh):
        c = lax.axis_index("c")
        s = lax.axis_index("s")
        _init_zero(zero_hbm, zsh, s)
        pltpu.sync_copy(ones_hbm, ones_v)
        plsc.subcore_barrier()
        base = (s * NC + c) * ept

        def blk(j, carry):
            off = base + j * K
            pltpu.sync_copy(dst_hbm.at[pl.ds(off, K)], idx_d)
            pltpu.sync_copy(ones_v, zsh.at[idx_d], add=True)
            return carry

        lax.fori_loop(0, nblk, blk, 0)
        plsc.subcore_barrier()
        _write_out(zsh, degz_hbm, c, s)

    return deg_kernel


def _make_scatter():
    ept = EA // (NC * NS)
    nblk = ept // K

    @functools.partial(
        pl.kernel, mesh=_mesh(),
        out_type=jax.ShapeDtypeStruct((NC, N, 128), jnp.float32),
        scratch_types=[
            pltpu.VMEM((K,), jnp.int32),
            pltpu.VMEM((K,), jnp.int32),
            pltpu.VMEM((K, 128), jnp.float32),
            pltpu.VMEM_SHARED((NP, 128), jnp.float32),
            pltpu.SemaphoreType.DMA,
        ],
    )
    def scat_kernel(src_hbm, dst_hbm, u_hbm, zero_hbm, z_hbm,
                    idx_s, idx_d, rows, zsh, sem):
        c = lax.axis_index("c")
        s = lax.axis_index("s")
        _init_zero(zero_hbm, zsh, s)
        plsc.subcore_barrier()
        base = (s * NC + c) * ept

        def blk(j, carry):
            off = base + j * K
            pltpu.sync_copy(src_hbm.at[pl.ds(off, K)], idx_s)
            pltpu.sync_copy(dst_hbm.at[pl.ds(off, K)], idx_d)
            pltpu.async_copy(u_hbm.at[idx_s], rows, sem).wait()
            pltpu.sync_copy(rows, zsh.at[idx_d], add=True)
            return carry

        lax.fori_loop(0, nblk, blk, 0)
        plsc.subcore_barrier()
        _write_out(zsh, z_hbm, c, s)

    return scat_kernel


_DEG = _make_deg()
_SCAT = _make_scatter()


def _tc_u1(x, degz):
    def body(x_ref, dz_ref, u_ref, dinv_ref):
        dz = dz_ref[...]
        dinv = lax.rsqrt(dz[0][:, 0:1] + dz[1][:, 0:1])
        u_ref[...] = x_ref[...] * dinv
        dinv_ref[...] = jnp.broadcast_to(dinv, (BM, 8))

    return pl.pallas_call(
        body,
        grid=(GRID,),
        in_specs=[pl.BlockSpec((BM, 128), lambda i: (i, 0)),
                  pl.BlockSpec((2, BM, 128), lambda i: (0, i, 0))],
        out_specs=[pl.BlockSpec((BM, 128), lambda i: (i, 0)),
                   pl.BlockSpec((BM, 8), lambda i: (i, 0))],
        out_shape=[jax.ShapeDtypeStruct((N, 128), jnp.float32),
                   jax.ShapeDtypeStruct((N, 8), jnp.float32)],
    )(x, degz)


def _tc_layer(z, dinv8, W, b, g, be, rm, rv):
    def body(z_ref, dinv_ref, w_ref, b_ref, g_ref, be_ref, rm_ref, rv_ref,
             u_ref):
        dinv = dinv_ref[:, 0:1]
        mm = jnp.dot(z_ref[0] + z_ref[1], w_ref[...],
                     preferred_element_type=jnp.float32)
        pre = mm * dinv + b_ref[...]
        scale = g_ref[...] * lax.rsqrt(rv_ref[...] + EPS)
        h = jnp.maximum((pre - rm_ref[...]) * scale + be_ref[...], 0.0)
        u_ref[...] = h * dinv

    pspec = pl.BlockSpec((1, 128), lambda i: (0, 0))
    return pl.pallas_call(
        body,
        grid=(GRID,),
        in_specs=[pl.BlockSpec((2, BM, 128), lambda i: (0, i, 0)),
                  pl.BlockSpec((BM, 8), lambda i: (i, 0)),
                  pl.BlockSpec((128, 128), lambda i: (0, 0)),
                  pspec, pspec, pspec, pspec, pspec],
        out_specs=pl.BlockSpec((BM, 128), lambda i: (i, 0)),
        out_shape=jax.ShapeDtypeStruct((N, 128), jnp.float32),
    )(z, dinv8, W, b, g, be, rm, rv)


def _tc_final(z3, dinv8, W3, b3, batch3):
    def body(z_ref, dinv_ref, w_ref, b_ref, bat_ref, out_ref, acc_ref, cnt_ref):
        i = pl.program_id(0)

        @pl.when(i == 0)
        def _():
            acc_ref[...] = jnp.zeros_like(acc_ref)
            cnt_ref[...] = jnp.zeros_like(cnt_ref)

        dinv = dinv_ref[:, 0:1]
        mm = jnp.dot(z_ref[0] + z_ref[1], w_ref[...],
                     preferred_element_type=jnp.float32)
        h = mm * dinv + b_ref[...]
        bidx = bat_ref[0, 0, :]
        oh = (bidx[:, None] == lax.broadcasted_iota(jnp.int32, (BM, G), 1)
              ).astype(jnp.float32)
        dn = (((0,), (0,)), ((), ()))
        acc_ref[...] += lax.dot_general(
            oh, h, dn, preferred_element_type=jnp.float32)
        cnt_ref[...] += lax.dot_general(
            oh, jnp.ones((BM, 128), jnp.float32), dn,
            preferred_element_type=jnp.float32)

        @pl.when(i == GRID - 1)
        def _():
            cnt = jnp.maximum(cnt_ref[:, 0:1], 1.0)
            out_ref[...] = acc_ref[...] / cnt

    return pl.pallas_call(
        body,
        grid=(GRID,),
        in_specs=[pl.BlockSpec((2, BM, 128), lambda i: (0, i, 0)),
                  pl.BlockSpec((BM, 8), lambda i: (i, 0)),
                  pl.BlockSpec((128, 256), lambda i: (0, 0)),
                  pl.BlockSpec((1, 256), lambda i: (0, 0)),
                  pl.BlockSpec((1, 1, BM), lambda i: (i, 0, 0))],
        out_specs=pl.BlockSpec((G, 256), lambda i: (0, 0)),
        out_shape=jax.ShapeDtypeStruct((G, 256), jnp.float32),
        scratch_shapes=[pltpu.VMEM((G, 256), jnp.float32),
                        pltpu.VMEM((G, 128), jnp.float32)],
    )(z3, dinv8, W3, b3, batch3)


def kernel(x, edge_index, batch, W1, b1, g1, be1, rm1, rv1,
           W2, b2, g2, be2, rm2, rv2, W3, b3):
    ei = edge_index.astype(jnp.int32)
    loops = jnp.arange(N, dtype=jnp.int32)
    pad = EA - E - N
    src_a = jnp.concatenate([ei[0], loops, jnp.zeros((pad,), jnp.int32)])
    dst_a = jnp.concatenate([ei[1], loops, jnp.full((pad,), N, jnp.int32)])
    batch3 = batch.astype(jnp.int32).reshape(GRID, 1, BM)
    ones_k = jnp.ones((K, 128), jnp.float32)
    zeros_np = jnp.zeros((NP, 128), jnp.float32)

    def row(a):
        return a.reshape(1, -1)

    degz = _DEG(dst_a, ones_k, zeros_np)
    u1, dinv8 = _tc_u1(x, degz)
    z1 = _SCAT(src_a, dst_a, u1, zeros_np)
    u2 = _tc_layer(z1, dinv8, W1, row(b1), row(g1), row(be1), row(rm1), row(rv1))
    z2 = _SCAT(src_a, dst_a, u2, zeros_np)
    u3 = _tc_layer(z2, dinv8, W2, row(b2), row(g2), row(be2), row(rm2), row(rv2))
    z3 = _SCAT(src_a, dst_a, u3, zeros_np)
    return _tc_final(z3, dinv8, W3, row(b3), batch3)


# speedup vs baseline: 2.4636x; 1.3163x over previous
import functools

import jax
import jax.numpy as jnp
from jax import lax
from jax.experimental import pallas as pl
from jax.experimental.pallas import tpu as pltpu
from jax.experimental.pallas import tpu_sc as plsc

N = 10000
E = 320000
G = 64
EPS = 1e-5
NC, NS = 2, 16
BM = 400
GRID = N // BM
K = 128
EA = 331776
NP = N + 16
STRIPE = 624


def _mesh():
    return plsc.VectorSubcoreMesh(
        core_axis_name="c", subcore_axis_name="s",
        num_cores=NC, num_subcores=NS)


def _init_zero(zero_hbm, zsh, s):
    r0 = s * STRIPE
    pltpu.sync_copy(zero_hbm.at[pl.ds(r0, STRIPE)], zsh.at[pl.ds(r0, STRIPE)])

    @pl.when(s == NS - 1)
    def _():
        t0 = NS * STRIPE
        pltpu.sync_copy(zero_hbm.at[pl.ds(t0, NP - t0)], zsh.at[pl.ds(t0, NP - t0)])


def _write_out(zsh, out_hbm, c, s):
    r0 = s * STRIPE
    pltpu.sync_copy(zsh.at[pl.ds(r0, STRIPE)], out_hbm.at[c, pl.ds(r0, STRIPE)])

    @pl.when(s == NS - 1)
    def _():
        t0 = NS * STRIPE
        pltpu.sync_copy(zsh.at[pl.ds(t0, N - t0)], out_hbm.at[c, pl.ds(t0, N - t0)])


def _make_deg():
    ept = EA // (NC * NS)
    nblk = ept // K

    @functools.partial(
        pl.kernel, mesh=_mesh(),
        out_type=jax.ShapeDtypeStruct((NC, N, 128), jnp.float32),
        scratch_types=[
            pltpu.VMEM((K,), jnp.int32),
            pltpu.VMEM((K, 128), jnp.float32),
            pltpu.VMEM_SHARED((NP, 128), jnp.float32),
        ],
    )
    def deg_kernel(dst_hbm, ones_hbm, zero_hbm, degz_hbm, idx_d, ones_v, zsh):
        c = lax.axis_index("c")
        s = lax.axis_index("s")
        _init_zero(zero_hbm, zsh, s)
        pltpu.sync_copy(ones_hbm, ones_v)
        plsc.subcore_barrier()
        base = (s * NC + c) * ept

        def blk(j, carry):
            off = base + j * K
            pltpu.sync_copy(dst_hbm.at[pl.ds(off, K)], idx_d)
            pltpu.sync_copy(ones_v, zsh.at[idx_d], add=True)
            return carry

        lax.fori_loop(0, nblk, blk, 0)
        plsc.subcore_barrier()
        _write_out(zsh, degz_hbm, c, s)

    return deg_kernel


def _make_scatter():
    ept = EA // (NC * NS)
    nblk = ept // K

    @functools.partial(
        pl.kernel, mesh=_mesh(),
        out_type=jax.ShapeDtypeStruct((NC, N, 128), jnp.float32),
        scratch_types=[
            pltpu.VMEM((2, K), jnp.int32),
            pltpu.VMEM((2, K), jnp.int32),
            pltpu.VMEM((2, K, 128), jnp.float32),
            pltpu.VMEM_SHARED((NP, 128), jnp.float32),
            pltpu.SemaphoreType.DMA,
            pltpu.SemaphoreType.DMA,
        ],
    )
    def scat_kernel(src_hbm, dst_hbm, u_hbm, zero_hbm, z_hbm,
                    idx_s, idx_d, rows, zsh, sem0, sem1):
        sems = (sem0, sem1)
        c = lax.axis_index("c")
        s = lax.axis_index("s")
        _init_zero(zero_hbm, zsh, s)
        base = (s * NC + c) * ept

        def load_idx(kk, u):
            off = base + kk * K
            pltpu.sync_copy(src_hbm.at[pl.ds(off, K)], idx_s.at[u])
            pltpu.sync_copy(dst_hbm.at[pl.ds(off, K)], idx_d.at[u])

        load_idx(0, 0)
        plsc.subcore_barrier()
        pltpu.async_copy(u_hbm.at[idx_s.at[0]], rows.at[0], sems[0])

        def turn(kk, u):
            load_idx(kk + 1, 1 - u)
            pltpu.async_copy(u_hbm.at[idx_s.at[1 - u]], rows.at[1 - u],
                             sems[1 - u])
            pltpu.make_async_copy(u_hbm.at[idx_s.at[u]], rows.at[u],
                                  sems[u]).wait()
            pltpu.sync_copy(rows.at[u], zsh.at[idx_d.at[u]], add=True)

        def outer(j, carry):
            turn(2 * j, 0)
            turn(2 * j + 1, 1)
            return carry

        lax.fori_loop(0, (nblk - 1) // 2, outer, 0)
        ul = (nblk - 1) % 2
        pltpu.make_async_copy(u_hbm.at[idx_s.at[ul]], rows.at[ul],
                              sems[ul]).wait()
        pltpu.sync_copy(rows.at[ul], zsh.at[idx_d.at[ul]], add=True)
        plsc.subcore_barrier()
        _write_out(zsh, z_hbm, c, s)

    return scat_kernel


_DEG = _make_deg()
_SCAT = _make_scatter()


def _tc_u1(x, degz):
    def body(x_ref, dz_ref, u_ref, dinv_ref):
        dz = dz_ref[...]
        dinv = lax.rsqrt(dz[0][:, 0:1] + dz[1][:, 0:1])
        u_ref[...] = x_ref[...] * dinv
        dinv_ref[...] = jnp.broadcast_to(dinv, (BM, 8))

    return pl.pallas_call(
        body,
        grid=(GRID,),
        in_specs=[pl.BlockSpec((BM, 128), lambda i: (i, 0)),
                  pl.BlockSpec((2, BM, 128), lambda i: (0, i, 0))],
        out_specs=[pl.BlockSpec((BM, 128), lambda i: (i, 0)),
                   pl.BlockSpec((BM, 8), lambda i: (i, 0))],
        out_shape=[jax.ShapeDtypeStruct((N, 128), jnp.float32),
                   jax.ShapeDtypeStruct((N, 8), jnp.float32)],
    )(x, degz)


def _tc_layer(z, dinv8, W, b, g, be, rm, rv):
    def body(z_ref, dinv_ref, w_ref, b_ref, g_ref, be_ref, rm_ref, rv_ref,
             u_ref):
        dinv = dinv_ref[:, 0:1]
        mm = jnp.dot(z_ref[0] + z_ref[1], w_ref[...],
                     preferred_element_type=jnp.float32)
        pre = mm * dinv + b_ref[...]
        scale = g_ref[...] * lax.rsqrt(rv_ref[...] + EPS)
        h = jnp.maximum((pre - rm_ref[...]) * scale + be_ref[...], 0.0)
        u_ref[...] = h * dinv

    pspec = pl.BlockSpec((1, 128), lambda i: (0, 0))
    return pl.pallas_call(
        body,
        grid=(GRID,),
        in_specs=[pl.BlockSpec((2, BM, 128), lambda i: (0, i, 0)),
                  pl.BlockSpec((BM, 8), lambda i: (i, 0)),
                  pl.BlockSpec((128, 128), lambda i: (0, 0)),
                  pspec, pspec, pspec, pspec, pspec],
        out_specs=pl.BlockSpec((BM, 128), lambda i: (i, 0)),
        out_shape=jax.ShapeDtypeStruct((N, 128), jnp.float32),
    )(z, dinv8, W, b, g, be, rm, rv)


def _tc_final(z3, dinv8, W3, b3, batch3):
    def body(z_ref, dinv_ref, w_ref, b_ref, bat_ref, out_ref, acc_ref, cnt_ref):
        i = pl.program_id(0)

        @pl.when(i == 0)
        def _():
            acc_ref[...] = jnp.zeros_like(acc_ref)
            cnt_ref[...] = jnp.zeros_like(cnt_ref)

        dinv = dinv_ref[:, 0:1]
        mm = jnp.dot(z_ref[0] + z_ref[1], w_ref[...],
                     preferred_element_type=jnp.float32)
        h = mm * dinv + b_ref[...]
        bidx = bat_ref[0, 0, :]
        oh = (bidx[:, None] == lax.broadcasted_iota(jnp.int32, (BM, G), 1)
              ).astype(jnp.float32)
        dn = (((0,), (0,)), ((), ()))
        acc_ref[...] += lax.dot_general(
            oh, h, dn, preferred_element_type=jnp.float32)
        cnt_ref[...] += lax.dot_general(
            oh, jnp.ones((BM, 128), jnp.float32), dn,
            preferred_element_type=jnp.float32)

        @pl.when(i == GRID - 1)
        def _():
            cnt = jnp.maximum(cnt_ref[:, 0:1], 1.0)
            out_ref[...] = acc_ref[...] / cnt

    return pl.pallas_call(
        body,
        grid=(GRID,),
        in_specs=[pl.BlockSpec((2, BM, 128), lambda i: (0, i, 0)),
                  pl.BlockSpec((BM, 8), lambda i: (i, 0)),
                  pl.BlockSpec((128, 256), lambda i: (0, 0)),
                  pl.BlockSpec((1, 256), lambda i: (0, 0)),
                  pl.BlockSpec((1, 1, BM), lambda i: (i, 0, 0))],
        out_specs=pl.BlockSpec((G, 256), lambda i: (0, 0)),
        out_shape=jax.ShapeDtypeStruct((G, 256), jnp.float32),
        scratch_shapes=[pltpu.VMEM((G, 256), jnp.float32),
                        pltpu.VMEM((G, 128), jnp.float32)],
    )(z3, dinv8, W3, b3, batch3)


def kernel(x, edge_index, batch, W1, b1, g1, be1, rm1, rv1,
           W2, b2, g2, be2, rm2, rv2, W3, b3):
    ei = edge_index.astype(jnp.int32)
    loops = jnp.arange(N, dtype=jnp.int32)
    pad = EA - E - N
    src_a = jnp.concatenate([ei[0], loops, jnp.zeros((pad,), jnp.int32)])
    dst_a = jnp.concatenate([ei[1], loops, jnp.full((pad,), N, jnp.int32)])
    batch3 = batch.astype(jnp.int32).reshape(GRID, 1, BM)
    ones_k = jnp.ones((K, 128), jnp.float32)
    zeros_np = jnp.zeros((NP, 128), jnp.float32)

    def row(a):
        return a.reshape(1, -1)

    degz = _DEG(dst_a, ones_k, zeros_np)
    u1, dinv8 = _tc_u1(x, degz)
    z1 = _SCAT(src_a, dst_a, u1, zeros_np)
    u2 = _tc_layer(z1, dinv8, W1, row(b1), row(g1), row(be1), row(rm1), row(rv1))
    z2 = _SCAT(src_a, dst_a, u2, zeros_np)
    u3 = _tc_layer(z2, dinv8, W2, row(b2), row(g2), row(be2), row(rm2), row(rv2))
    z3 = _SCAT(src_a, dst_a, u3, zeros_np)
    return _tc_final(z3, dinv8, W3, row(b3), batch3)


# speedup vs baseline: 3.0894x; 1.2540x over previous
import functools

import jax
import jax.numpy as jnp
from jax import lax
from jax.experimental import pallas as pl
from jax.experimental.pallas import tpu as pltpu
from jax.experimental.pallas import tpu_sc as plsc

N = 10000
E = 320000
G = 64
EPS = 1e-5
NC, NS = 2, 16
BM = 400
GRID = N // BM
K = 128
EA = 331776
NP = N + 16
STRIPE = 624


def _mesh():
    return plsc.VectorSubcoreMesh(
        core_axis_name="c", subcore_axis_name="s",
        num_cores=NC, num_subcores=NS)


def _init_zero(zero_hbm, zsh, s):
    r0 = s * STRIPE
    pltpu.sync_copy(zero_hbm.at[pl.ds(r0, STRIPE)], zsh.at[pl.ds(r0, STRIPE)])

    @pl.when(s == NS - 1)
    def _():
        t0 = NS * STRIPE
        pltpu.sync_copy(zero_hbm.at[pl.ds(t0, NP - t0)], zsh.at[pl.ds(t0, NP - t0)])


def _write_out(zsh, out_hbm, c, s):
    r0 = s * STRIPE
    pltpu.sync_copy(zsh.at[pl.ds(r0, STRIPE)], out_hbm.at[c, pl.ds(r0, STRIPE)])

    @pl.when(s == NS - 1)
    def _():
        t0 = NS * STRIPE
        pltpu.sync_copy(zsh.at[pl.ds(t0, N - t0)], out_hbm.at[c, pl.ds(t0, N - t0)])


def _make_deg():
    ept = EA // (NC * NS)
    nblk = ept // K

    @functools.partial(
        pl.kernel, mesh=_mesh(),
        out_type=jax.ShapeDtypeStruct((NC, N, 128), jnp.float32),
        scratch_types=[
            pltpu.VMEM((3, K), jnp.int32),
            pltpu.VMEM((K, 128), jnp.float32),
            pltpu.VMEM_SHARED((NP, 128), jnp.float32),
        ] + [pltpu.SemaphoreType.DMA] * 6,
    )
    def deg_kernel(dst_hbm, ones_hbm, zero_hbm, degz_hbm, idx_d, ones_v, zsh,
                   *sems):
        sem_i = sems[0:3]
        sem_s = sems[3:6]
        c = lax.axis_index("c")
        s = lax.axis_index("s")
        _init_zero(zero_hbm, zsh, s)
        pltpu.sync_copy(ones_hbm, ones_v)
        base = (s * NC + c) * ept

        def issue_idx(kk, u):
            pltpu.async_copy(dst_hbm.at[pl.ds(base + kk * K, K)],
                             idx_d.at[u], sem_i[u])

        def wait_idx(u):
            pltpu.make_async_copy(dst_hbm.at[pl.ds(base, K)],
                                  idx_d.at[u], sem_i[u]).wait()

        def wait_scat(u):
            pltpu.make_async_copy(ones_v, zsh.at[idx_d.at[u]],
                                  sem_s[u]).wait()

        issue_idx(0, 0)
        issue_idx(1, 1)
        plsc.subcore_barrier()

        def turn(kk, u):
            wait_idx(u)
            pltpu.async_copy(ones_v, zsh.at[idx_d.at[u]], sem_s[u], add=True)

            @pl.when(kk >= 1)
            def _():
                wait_scat((u + 2) % 3)

            @pl.when(kk + 2 < nblk)
            def _():
                issue_idx(kk + 2, (u + 2) % 3)

        def outer(j, carry):
            for u in range(3):
                turn(3 * j + u, u)
            return carry

        lax.fori_loop(0, nblk // 3, outer, 0)
        wait_scat((nblk - 1) % 3)
        plsc.subcore_barrier()
        _write_out(zsh, degz_hbm, c, s)

    return deg_kernel


def _make_scatter():
    ept = EA // (NC * NS)
    nblk = ept // K

    @functools.partial(
        pl.kernel, mesh=_mesh(),
        out_type=jax.ShapeDtypeStruct((NC, N, 128), jnp.float32),
        scratch_types=[
            pltpu.VMEM((2, K), jnp.int32),
            pltpu.VMEM((2, K), jnp.int32),
            pltpu.VMEM((2, K, 128), jnp.float32),
            pltpu.VMEM_SHARED((NP, 128), jnp.float32),
        ] + [pltpu.SemaphoreType.DMA] * 4,
    )
    def scat_kernel(src_hbm, dst_hbm, u_hbm, zero_hbm, z_hbm,
                    idx_s, idx_d, rows, zsh, semg0, semg1, semi0, semi1):
        sem_g = (semg0, semg1)
        sem_i = (semi0, semi1)
        c = lax.axis_index("c")
        s = lax.axis_index("s")
        _init_zero(zero_hbm, zsh, s)
        base = (s * NC + c) * ept

        def issue_idx(kk, u):
            off = base + kk * K
            pltpu.async_copy(src_hbm.at[pl.ds(off, K)], idx_s.at[u], sem_i[u])
            pltpu.async_copy(dst_hbm.at[pl.ds(off, K)], idx_d.at[u], sem_i[u])

        def wait_idx(u):
            pltpu.make_async_copy(src_hbm.at[pl.ds(base, K)],
                                  idx_s.at[u], sem_i[u]).wait()
            pltpu.make_async_copy(dst_hbm.at[pl.ds(base, K)],
                                  idx_d.at[u], sem_i[u]).wait()

        issue_idx(0, 0)
        issue_idx(1, 1)
        plsc.subcore_barrier()
        wait_idx(0)
        pltpu.async_copy(u_hbm.at[idx_s.at[0]], rows.at[0], sem_g[0])

        def turn(kk, u):
            wait_idx(1 - u)
            pltpu.async_copy(u_hbm.at[idx_s.at[1 - u]], rows.at[1 - u],
                             sem_g[1 - u])
            pltpu.make_async_copy(u_hbm.at[idx_s.at[u]], rows.at[u],
                                  sem_g[u]).wait()
            pltpu.sync_copy(rows.at[u], zsh.at[idx_d.at[u]], add=True)
            issue_idx(kk + 2, u)

        def outer(j, carry):
            turn(2 * j, 0)
            turn(2 * j + 1, 1)
            return carry

        lax.fori_loop(0, (nblk - 1) // 2, outer, 0)
        ul = (nblk - 1) % 2
        pltpu.make_async_copy(u_hbm.at[idx_s.at[ul]], rows.at[ul],
                              sem_g[ul]).wait()
        pltpu.sync_copy(rows.at[ul], zsh.at[idx_d.at[ul]], add=True)
        wait_idx(1 - ul)
        plsc.subcore_barrier()
        _write_out(zsh, z_hbm, c, s)

    return scat_kernel


_DEG = _make_deg()
_SCAT = _make_scatter()


def _tc_u1(x, degz):
    def body(x_ref, dz_ref, u_ref, dinv_ref):
        dz = dz_ref[...]
        dinv = lax.rsqrt(dz[0][:, 0:1] + dz[1][:, 0:1])
        u_ref[...] = x_ref[...] * dinv
        dinv_ref[...] = jnp.broadcast_to(dinv, (BM, 8))

    return pl.pallas_call(
        body,
        grid=(GRID,),
        in_specs=[pl.BlockSpec((BM, 128), lambda i: (i, 0)),
                  pl.BlockSpec((2, BM, 128), lambda i: (0, i, 0))],
        out_specs=[pl.BlockSpec((BM, 128), lambda i: (i, 0)),
                   pl.BlockSpec((BM, 8), lambda i: (i, 0))],
        out_shape=[jax.ShapeDtypeStruct((N, 128), jnp.float32),
                   jax.ShapeDtypeStruct((N, 8), jnp.float32)],
    )(x, degz)


def _tc_layer(z, dinv8, W, b, g, be, rm, rv):
    def body(z_ref, dinv_ref, w_ref, b_ref, g_ref, be_ref, rm_ref, rv_ref,
             u_ref):
        dinv = dinv_ref[:, 0:1]
        mm = jnp.dot(z_ref[0] + z_ref[1], w_ref[...],
                     preferred_element_type=jnp.float32)
        pre = mm * dinv + b_ref[...]
        scale = g_ref[...] * lax.rsqrt(rv_ref[...] + EPS)
        h = jnp.maximum((pre - rm_ref[...]) * scale + be_ref[...], 0.0)
        u_ref[...] = h * dinv

    pspec = pl.BlockSpec((1, 128), lambda i: (0, 0))
    return pl.pallas_call(
        body,
        grid=(GRID,),
        in_specs=[pl.BlockSpec((2, BM, 128), lambda i: (0, i, 0)),
                  pl.BlockSpec((BM, 8), lambda i: (i, 0)),
                  pl.BlockSpec((128, 128), lambda i: (0, 0)),
                  pspec, pspec, pspec, pspec, pspec],
        out_specs=pl.BlockSpec((BM, 128), lambda i: (i, 0)),
        out_shape=jax.ShapeDtypeStruct((N, 128), jnp.float32),
    )(z, dinv8, W, b, g, be, rm, rv)


def _tc_final(z3, dinv8, W3, b3, batch3):
    def body(z_ref, dinv_ref, w_ref, b_ref, bat_ref, out_ref, acc_ref, cnt_ref):
        i = pl.program_id(0)

        @pl.when(i == 0)
        def _():
            acc_ref[...] = jnp.zeros_like(acc_ref)
            cnt_ref[...] = jnp.zeros_like(cnt_ref)

        dinv = dinv_ref[:, 0:1]
        mm = jnp.dot(z_ref[0] + z_ref[1], w_ref[...],
                     preferred_element_type=jnp.float32)
        h = mm * dinv + b_ref[...]
        bidx = bat_ref[0, 0, :]
        oh = (bidx[:, None] == lax.broadcasted_iota(jnp.int32, (BM, G), 1)
              ).astype(jnp.float32)
        dn = (((0,), (0,)), ((), ()))
        acc_ref[...] += lax.dot_general(
            oh, h, dn, preferred_element_type=jnp.float32)
        cnt_ref[...] += lax.dot_general(
            oh, jnp.ones((BM, 128), jnp.float32), dn,
            preferred_element_type=jnp.float32)

        @pl.when(i == GRID - 1)
        def _():
            cnt = jnp.maximum(cnt_ref[:, 0:1], 1.0)
            out_ref[...] = acc_ref[...] / cnt

    return pl.pallas_call(
        body,
        grid=(GRID,),
        in_specs=[pl.BlockSpec((2, BM, 128), lambda i: (0, i, 0)),
                  pl.BlockSpec((BM, 8), lambda i: (i, 0)),
                  pl.BlockSpec((128, 256), lambda i: (0, 0)),
                  pl.BlockSpec((1, 256), lambda i: (0, 0)),
                  pl.BlockSpec((1, 1, BM), lambda i: (i, 0, 0))],
        out_specs=pl.BlockSpec((G, 256), lambda i: (0, 0)),
        out_shape=jax.ShapeDtypeStruct((G, 256), jnp.float32),
        scratch_shapes=[pltpu.VMEM((G, 256), jnp.float32),
                        pltpu.VMEM((G, 128), jnp.float32)],
    )(z3, dinv8, W3, b3, batch3)


def kernel(x, edge_index, batch, W1, b1, g1, be1, rm1, rv1,
           W2, b2, g2, be2, rm2, rv2, W3, b3):
    ei = edge_index.astype(jnp.int32)
    loops = jnp.arange(N, dtype=jnp.int32)
    pad = EA - E - N + K
    src_a = jnp.concatenate([ei[0], loops, jnp.zeros((pad,), jnp.int32)])
    dst_a = jnp.concatenate([ei[1], loops, jnp.full((pad,), N, jnp.int32)])
    batch3 = batch.astype(jnp.int32).reshape(GRID, 1, BM)
    ones_k = jnp.ones((K, 128), jnp.float32)
    zeros_np = jnp.zeros((NP, 128), jnp.float32)

    def row(a):
        return a.reshape(1, -1)

    degz = _DEG(dst_a, ones_k, zeros_np)
    u1, dinv8 = _tc_u1(x, degz)
    z1 = _SCAT(src_a, dst_a, u1, zeros_np)
    u2 = _tc_layer(z1, dinv8, W1, row(b1), row(g1), row(be1), row(rm1), row(rv1))
    z2 = _SCAT(src_a, dst_a, u2, zeros_np)
    u3 = _tc_layer(z2, dinv8, W2, row(b2), row(g2), row(be2), row(rm2), row(rv2))
    z3 = _SCAT(src_a, dst_a, u3, zeros_np)
    return _tc_final(z3, dinv8, W3, row(b3), batch3)


# speedup vs baseline: 3.2627x; 1.0561x over previous
import functools

import jax
import jax.numpy as jnp
from jax import lax
from jax.experimental import pallas as pl
from jax.experimental.pallas import tpu as pltpu
from jax.experimental.pallas import tpu_sc as plsc

N = 10000
E = 320000
G = 64
EPS = 1e-5
NC, NS = 2, 16
BM = 2000
GRID = N // BM
K = 128
EA = 331776
NP = N + 16
STRIPE = 624


def _mesh():
    return plsc.VectorSubcoreMesh(
        core_axis_name="c", subcore_axis_name="s",
        num_cores=NC, num_subcores=NS)


def _init_zero(zero_hbm, zsh, s):
    r0 = s * STRIPE
    pltpu.sync_copy(zero_hbm.at[pl.ds(r0, STRIPE)], zsh.at[pl.ds(r0, STRIPE)])

    @pl.when(s == NS - 1)
    def _():
        t0 = NS * STRIPE
        pltpu.sync_copy(zero_hbm.at[pl.ds(t0, NP - t0)], zsh.at[pl.ds(t0, NP - t0)])


def _write_out(zsh, out_hbm, c, s):
    r0 = s * STRIPE
    pltpu.sync_copy(zsh.at[pl.ds(r0, STRIPE)], out_hbm.at[c, pl.ds(r0, STRIPE)])

    @pl.when(s == NS - 1)
    def _():
        t0 = NS * STRIPE
        pltpu.sync_copy(zsh.at[pl.ds(t0, N - t0)], out_hbm.at[c, pl.ds(t0, N - t0)])


def _make_deg():
    ept = EA // (NC * NS)
    nblk = ept // K

    @functools.partial(
        pl.kernel, mesh=_mesh(),
        out_type=jax.ShapeDtypeStruct((NC, N, 128), jnp.float32),
        scratch_types=[
            pltpu.VMEM((3, K), jnp.int32),
            pltpu.VMEM((K, 128), jnp.float32),
            pltpu.VMEM_SHARED((NP, 128), jnp.float32),
        ] + [pltpu.SemaphoreType.DMA] * 6,
    )
    def deg_kernel(dst_hbm, ones_hbm, zero_hbm, degz_hbm, idx_d, ones_v, zsh,
                   *sems):
        sem_i = sems[0:3]
        sem_s = sems[3:6]
        c = lax.axis_index("c")
        s = lax.axis_index("s")
        _init_zero(zero_hbm, zsh, s)
        pltpu.sync_copy(ones_hbm, ones_v)
        base = (s * NC + c) * ept

        def issue_idx(kk, u):
            pltpu.async_copy(dst_hbm.at[pl.ds(base + kk * K, K)],
                             idx_d.at[u], sem_i[u])

        def wait_idx(u):
            pltpu.make_async_copy(dst_hbm.at[pl.ds(base, K)],
                                  idx_d.at[u], sem_i[u]).wait()

        def wait_scat(u):
            pltpu.make_async_copy(ones_v, zsh.at[idx_d.at[u]],
                                  sem_s[u]).wait()

        issue_idx(0, 0)
        issue_idx(1, 1)
        plsc.subcore_barrier()

        def turn(kk, u):
            wait_idx(u)
            pltpu.async_copy(ones_v, zsh.at[idx_d.at[u]], sem_s[u], add=True)

            @pl.when(kk >= 1)
            def _():
                wait_scat((u + 2) % 3)

            @pl.when(kk + 2 < nblk)
            def _():
                issue_idx(kk + 2, (u + 2) % 3)

        def outer(j, carry):
            for u in range(3):
                turn(3 * j + u, u)
            return carry

        lax.fori_loop(0, nblk // 3, outer, 0)
        wait_scat((nblk - 1) % 3)
        plsc.subcore_barrier()
        _write_out(zsh, degz_hbm, c, s)

    return deg_kernel


def _make_scatter():
    ept = EA // (NC * NS)
    nblk = ept // K

    @functools.partial(
        pl.kernel, mesh=_mesh(),
        out_type=jax.ShapeDtypeStruct((NC, N, 128), jnp.float32),
        scratch_types=[
            pltpu.VMEM((2, K), jnp.int32),
            pltpu.VMEM((2, K), jnp.int32),
            pltpu.VMEM((2, K, 128), jnp.float32),
            pltpu.VMEM_SHARED((NP, 128), jnp.float32),
        ] + [pltpu.SemaphoreType.DMA] * 4,
    )
    def scat_kernel(src_hbm, dst_hbm, u_hbm, zero_hbm, z_hbm,
                    idx_s, idx_d, rows, zsh, semg0, semg1, semi0, semi1):
        sem_g = (semg0, semg1)
        sem_i = (semi0, semi1)
        c = lax.axis_index("c")
        s = lax.axis_index("s")
        _init_zero(zero_hbm, zsh, s)
        base = (s * NC + c) * ept

        def issue_idx(kk, u):
            off = base + kk * K
            pltpu.async_copy(src_hbm.at[pl.ds(off, K)], idx_s.at[u], sem_i[u])
            pltpu.async_copy(dst_hbm.at[pl.ds(off, K)], idx_d.at[u], sem_i[u])

        def wait_idx(u):
            pltpu.make_async_copy(src_hbm.at[pl.ds(base, K)],
                                  idx_s.at[u], sem_i[u]).wait()
            pltpu.make_async_copy(dst_hbm.at[pl.ds(base, K)],
                                  idx_d.at[u], sem_i[u]).wait()

        issue_idx(0, 0)
        issue_idx(1, 1)
        plsc.subcore_barrier()
        wait_idx(0)
        pltpu.async_copy(u_hbm.at[idx_s.at[0]], rows.at[0], sem_g[0])

        def turn(kk, u):
            wait_idx(1 - u)
            pltpu.async_copy(u_hbm.at[idx_s.at[1 - u]], rows.at[1 - u],
                             sem_g[1 - u])
            pltpu.make_async_copy(u_hbm.at[idx_s.at[u]], rows.at[u],
                                  sem_g[u]).wait()
            pltpu.sync_copy(rows.at[u], zsh.at[idx_d.at[u]], add=True)
            issue_idx(kk + 2, u)

        def outer(j, carry):
            turn(2 * j, 0)
            turn(2 * j + 1, 1)
            return carry

        lax.fori_loop(0, (nblk - 1) // 2, outer, 0)
        ul = (nblk - 1) % 2
        pltpu.make_async_copy(u_hbm.at[idx_s.at[ul]], rows.at[ul],
                              sem_g[ul]).wait()
        pltpu.sync_copy(rows.at[ul], zsh.at[idx_d.at[ul]], add=True)
        wait_idx(1 - ul)
        plsc.subcore_barrier()
        _write_out(zsh, z_hbm, c, s)

    return scat_kernel


_DEG = _make_deg()
_SCAT = _make_scatter()


def _tc_u1(x, degz):
    def body(x_ref, dz_ref, u_ref, dinv_ref):
        dz = dz_ref[...]
        dinv = lax.rsqrt(dz[0][:, 0:1] + dz[1][:, 0:1])
        u_ref[...] = x_ref[...] * dinv
        dinv_ref[...] = jnp.broadcast_to(dinv, (BM, 8))

    return pl.pallas_call(
        body,
        grid=(GRID,),
        in_specs=[pl.BlockSpec((BM, 128), lambda i: (i, 0)),
                  pl.BlockSpec((2, BM, 128), lambda i: (0, i, 0))],
        out_specs=[pl.BlockSpec((BM, 128), lambda i: (i, 0)),
                   pl.BlockSpec((BM, 8), lambda i: (i, 0))],
        out_shape=[jax.ShapeDtypeStruct((N, 128), jnp.float32),
                   jax.ShapeDtypeStruct((N, 8), jnp.float32)],
    )(x, degz)


def _tc_layer(z, dinv8, W, b, g, be, rm, rv):
    def body(z_ref, dinv_ref, w_ref, b_ref, g_ref, be_ref, rm_ref, rv_ref,
             u_ref):
        dinv = dinv_ref[:, 0:1]
        mm = jnp.dot(z_ref[0] + z_ref[1], w_ref[...],
                     preferred_element_type=jnp.float32)
        pre = mm * dinv + b_ref[...]
        scale = g_ref[...] * lax.rsqrt(rv_ref[...] + EPS)
        h = jnp.maximum((pre - rm_ref[...]) * scale + be_ref[...], 0.0)
        u_ref[...] = h * dinv

    pspec = pl.BlockSpec((1, 128), lambda i: (0, 0))
    return pl.pallas_call(
        body,
        grid=(GRID,),
        in_specs=[pl.BlockSpec((2, BM, 128), lambda i: (0, i, 0)),
                  pl.BlockSpec((BM, 8), lambda i: (i, 0)),
                  pl.BlockSpec((128, 128), lambda i: (0, 0)),
                  pspec, pspec, pspec, pspec, pspec],
        out_specs=pl.BlockSpec((BM, 128), lambda i: (i, 0)),
        out_shape=jax.ShapeDtypeStruct((N, 128), jnp.float32),
    )(z, dinv8, W, b, g, be, rm, rv)


def _tc_final(z3, dinv8, W3, b3, batch3):
    def body(z_ref, dinv_ref, w_ref, b_ref, bat_ref, out_ref, acc_ref, cnt_ref):
        i = pl.program_id(0)

        @pl.when(i == 0)
        def _():
            acc_ref[...] = jnp.zeros_like(acc_ref)
            cnt_ref[...] = jnp.zeros_like(cnt_ref)

        dinv = dinv_ref[:, 0:1]
        mm = jnp.dot(z_ref[0] + z_ref[1], w_ref[...],
                     preferred_element_type=jnp.float32)
        h = mm * dinv + b_ref[...]
        bidx = bat_ref[0, 0, :]
        oh = (bidx[:, None] == lax.broadcasted_iota(jnp.int32, (BM, G), 1)
              ).astype(jnp.float32)
        dn = (((0,), (0,)), ((), ()))
        acc_ref[...] += lax.dot_general(
            oh, h, dn, preferred_element_type=jnp.float32)
        cnt_ref[...] += lax.dot_general(
            oh, jnp.ones((BM, 128), jnp.float32), dn,
            preferred_element_type=jnp.float32)

        @pl.when(i == GRID - 1)
        def _():
            cnt = jnp.maximum(cnt_ref[:, 0:1], 1.0)
            out_ref[...] = acc_ref[...] / cnt

    return pl.pallas_call(
        body,
        grid=(GRID,),
        in_specs=[pl.BlockSpec((2, BM, 128), lambda i: (0, i, 0)),
                  pl.BlockSpec((BM, 8), lambda i: (i, 0)),
                  pl.BlockSpec((128, 256), lambda i: (0, 0)),
                  pl.BlockSpec((1, 256), lambda i: (0, 0)),
                  pl.BlockSpec((1, 1, BM), lambda i: (i, 0, 0))],
        out_specs=pl.BlockSpec((G, 256), lambda i: (0, 0)),
        out_shape=jax.ShapeDtypeStruct((G, 256), jnp.float32),
        scratch_shapes=[pltpu.VMEM((G, 256), jnp.float32),
                        pltpu.VMEM((G, 128), jnp.float32)],
    )(z3, dinv8, W3, b3, batch3)


def kernel(x, edge_index, batch, W1, b1, g1, be1, rm1, rv1,
           W2, b2, g2, be2, rm2, rv2, W3, b3):
    ei = edge_index.astype(jnp.int32)
    loops = jnp.arange(N, dtype=jnp.int32)
    pad = EA - E - N + K
    src_a = jnp.concatenate([ei[0], loops, jnp.zeros((pad,), jnp.int32)])
    dst_a = jnp.concatenate([ei[1], loops, jnp.full((pad,), N, jnp.int32)])
    batch3 = batch.astype(jnp.int32).reshape(GRID, 1, BM)
    ones_k = jnp.ones((K, 128), jnp.float32)
    zeros_np = jnp.zeros((NP, 128), jnp.float32)

    def row(a):
        return a.reshape(1, -1)

    degz = _DEG(dst_a, ones_k, zeros_np)
    u1, dinv8 = _tc_u1(x, degz)
    z1 = _SCAT(src_a, dst_a, u1, zeros_np)
    u2 = _tc_layer(z1, dinv8, W1, row(b1), row(g1), row(be1), row(rm1), row(rv1))
    z2 = _SCAT(src_a, dst_a, u2, zeros_np)
    u3 = _tc_layer(z2, dinv8, W2, row(b2), row(g2), row(be2), row(rm2), row(rv2))
    z3 = _SCAT(src_a, dst_a, u3, zeros_np)
    return _tc_final(z3, dinv8, W3, row(b3), batch3)
